# baseline (speedup 1.0000x reference)
import jax
import jax.numpy as jnp
from jax.experimental import pallas as pl
from jax.experimental.pallas import tpu as pltpu

N_TOKENS = 32768
HIDDEN = 768
N_EXPERTS = 64
TOP_K = 2
BLK = 1024
NBUF = 8
GRID = N_TOKENS // BLK
NEG = float("-inf")


def _router_kernel(x_hbm, w_ref, idx_ref, tw_ref, loss_ref, frac_ref,
                   probs_ref, xbuf, acc_ref, xsems):
    i = pl.program_id(0)
    nsteps = pl.num_programs(0)

    @pl.when(i == 0)
    def _prologue():
        for b in range(NBUF):
            pltpu.make_async_copy(
                x_hbm.at[pl.ds(b * BLK, BLK), :], xbuf.at[b], xsems.at[b]
            ).start()

    slot = jax.lax.rem(i, NBUF)
    pltpu.make_async_copy(
        x_hbm.at[pl.ds(i * BLK, BLK), :], xbuf.at[slot], xsems.at[slot]
    ).wait()

    v = jax.lax.dot_general(
        w_ref[...], xbuf[slot], (((1,), (1,)), ((), ())),
        preferred_element_type=jnp.float32)

    @pl.when(i + NBUF < nsteps)
    def _prefetch():
        pltpu.make_async_copy(
            x_hbm.at[pl.ds((i + NBUF) * BLK, BLK), :], xbuf.at[slot],
            xsems.at[slot]
        ).start()

    rows = jax.lax.broadcasted_iota(jnp.int32, v.shape, 0)
    m1 = jnp.max(v, axis=0, keepdims=True)
    idx1 = jnp.min(jnp.where(v == m1, rows, N_EXPERTS), axis=0,
                   keepdims=True)
    masked = jnp.where(rows == idx1, NEG, v)
    m2 = jnp.max(masked, axis=0, keepdims=True)
    idx2 = jnp.min(jnp.where(masked == m2, rows, N_EXPERTS), axis=0,
                   keepdims=True)

    e = jnp.exp(v - m1)
    probs = e / jnp.sum(e, axis=0, keepdims=True)
    probs_ref[...] = probs

    w1 = 1.0 / (1.0 + jnp.exp(m2 - m1))
    tw_ref[...] = jnp.concatenate([w1, 1.0 - w1], axis=0)
    idx_ref[...] = jnp.concatenate([idx1, idx2], axis=0)

    onehot = ((rows == idx1).astype(jnp.float32)
              + (rows == idx2).astype(jnp.float32))
    cnt = jnp.sum(onehot, axis=1, keepdims=True)
    ps = jnp.sum(probs, axis=1, keepdims=True)

    @pl.when(i == 0)
    def _init():
        acc_ref[...] = jnp.zeros_like(acc_ref)

    acc_ref[:, 0:1] += cnt
    acc_ref[:, 1:2] += ps

    @pl.when(i == nsteps - 1)
    def _fin():
        counts = acc_ref[:, 0:1]
        tot = acc_ref[:, 1:2]
        inv_n = 1.0 / N_TOKENS
        loss_ref[...] = (N_EXPERTS * inv_n * inv_n) * jnp.sum(
            counts * tot, keepdims=True)
        frac_ref[...] = jnp.transpose(counts) * inv_n


def kernel(x, W):
    idx_t, tw_t, loss, frac, probs_t = pl.pallas_call(
        _router_kernel,
        grid=(GRID,),
        in_specs=[
            pl.BlockSpec(memory_space=pl.ANY),
            pl.BlockSpec((N_EXPERTS, HIDDEN), lambda i: (0, 0)),
        ],
        out_specs=[
            pl.BlockSpec((TOP_K, BLK), lambda i: (0, i)),
            pl.BlockSpec((TOP_K, BLK), lambda i: (0, i)),
            pl.BlockSpec((1, 1), lambda i: (0, 0)),
            pl.BlockSpec((1, N_EXPERTS), lambda i: (0, 0)),
            pl.BlockSpec((N_EXPERTS, BLK), lambda i: (0, i)),
        ],
        out_shape=[
            jax.ShapeDtypeStruct((TOP_K, N_TOKENS), jnp.int32),
            jax.ShapeDtypeStruct((TOP_K, N_TOKENS), jnp.float32),
            jax.ShapeDtypeStruct((1, 1), jnp.float32),
            jax.ShapeDtypeStruct((1, N_EXPERTS), jnp.float32),
            jax.ShapeDtypeStruct((N_EXPERTS, N_TOKENS), jnp.float32),
        ],
        scratch_shapes=[
            pltpu.VMEM((NBUF, BLK, HIDDEN), jnp.float32),
            pltpu.VMEM((N_EXPERTS, 2), jnp.float32),
            pltpu.SemaphoreType.DMA((NBUF,)),
        ],
    )(x, W)
    return (jnp.transpose(idx_t), jnp.transpose(tw_t), loss[0, 0],
            frac[0], jnp.transpose(probs_t))


# speedup vs baseline: 1.0088x; 1.0088x over previous
import jax
import jax.numpy as jnp
from jax.experimental import pallas as pl
from jax.experimental.pallas import tpu as pltpu

N_TOKENS = 32768
HIDDEN = 768
N_EXPERTS = 64
TOP_K = 2
BLK = 2048
NBUF = 4
GRID = N_TOKENS // BLK
NEG = float("-inf")


def _router_kernel(x_hbm, w_ref, idx_ref, tw_ref, loss_ref, frac_ref,
                   probs_ref, xbuf, acc_ref, xsems):
    i = pl.program_id(0)
    nsteps = pl.num_programs(0)

    @pl.when(i == 0)
    def _prologue():
        for b in range(NBUF):
            pltpu.make_async_copy(
                x_hbm.at[pl.ds(b * BLK, BLK), :], xbuf.at[b], xsems.at[b]
            ).start()

    slot = jax.lax.rem(i, NBUF)
    pltpu.make_async_copy(
        x_hbm.at[pl.ds(i * BLK, BLK), :], xbuf.at[slot], xsems.at[slot]
    ).wait()

    v = jax.lax.dot_general(
        w_ref[...], xbuf[slot], (((1,), (1,)), ((), ())),
        preferred_element_type=jnp.float32)

    @pl.when(i + NBUF < nsteps)
    def _prefetch():
        pltpu.make_async_copy(
            x_hbm.at[pl.ds((i + NBUF) * BLK, BLK), :], xbuf.at[slot],
            xsems.at[slot]
        ).start()

    rows = jax.lax.broadcasted_iota(jnp.int32, v.shape, 0)
    m1 = jnp.max(v, axis=0, keepdims=True)
    idx1 = jnp.min(jnp.where(v == m1, rows, N_EXPERTS), axis=0,
                   keepdims=True)
    masked = jnp.where(rows == idx1, NEG, v)
    m2 = jnp.max(masked, axis=0, keepdims=True)
    idx2 = jnp.min(jnp.where(masked == m2, rows, N_EXPERTS), axis=0,
                   keepdims=True)

    e = jnp.exp(v - m1)
    probs = e / jnp.sum(e, axis=0, keepdims=True)
    probs_ref[...] = probs

    w1 = 1.0 / (1.0 + jnp.exp(m2 - m1))
    tw_ref[...] = jnp.concatenate([w1, 1.0 - w1], axis=0)
    idx_ref[...] = jnp.concatenate([idx1, idx2], axis=0)

    onehot = ((rows == idx1).astype(jnp.float32)
              + (rows == idx2).astype(jnp.float32))
    cnt = jnp.sum(onehot, axis=1, keepdims=True)
    ps = jnp.sum(probs, axis=1, keepdims=True)

    @pl.when(i == 0)
    def _init():
        acc_ref[...] = jnp.zeros_like(acc_ref)

    acc_ref[:, 0:1] += cnt
    acc_ref[:, 1:2] += ps

    @pl.when(i == nsteps - 1)
    def _fin():
        counts = acc_ref[:, 0:1]
        tot = acc_ref[:, 1:2]
        inv_n = 1.0 / N_TOKENS
        loss_ref[...] = (N_EXPERTS * inv_n * inv_n) * jnp.sum(
            counts * tot, keepdims=True)
        frac_ref[...] = jnp.transpose(counts) * inv_n


def kernel(x, W):
    idx_t, tw_t, loss, frac, probs_t = pl.pallas_call(
        _router_kernel,
        grid=(GRID,),
        in_specs=[
            pl.BlockSpec(memory_space=pl.ANY),
            pl.BlockSpec((N_EXPERTS, HIDDEN), lambda i: (0, 0)),
        ],
        out_specs=[
            pl.BlockSpec((TOP_K, BLK), lambda i: (0, i)),
            pl.BlockSpec((TOP_K, BLK), lambda i: (0, i)),
            pl.BlockSpec((1, 1), lambda i: (0, 0)),
            pl.BlockSpec((1, N_EXPERTS), lambda i: (0, 0)),
            pl.BlockSpec((N_EXPERTS, BLK), lambda i: (0, i)),
        ],
        out_shape=[
            jax.ShapeDtypeStruct((TOP_K, N_TOKENS), jnp.int32),
            jax.ShapeDtypeStruct((TOP_K, N_TOKENS), jnp.float32),
            jax.ShapeDtypeStruct((1, 1), jnp.float32),
            jax.ShapeDtypeStruct((1, N_EXPERTS), jnp.float32),
            jax.ShapeDtypeStruct((N_EXPERTS, N_TOKENS), jnp.float32),
        ],
        scratch_shapes=[
            pltpu.VMEM((NBUF, BLK, HIDDEN), jnp.float32),
            pltpu.VMEM((N_EXPERTS, 2), jnp.float32),
            pltpu.SemaphoreType.DMA((NBUF,)),
        ],
    )(x, W)
    return (jnp.transpose(idx_t), jnp.transpose(tw_t), loss[0, 0],
            frac[0], jnp.transpose(probs_t))
